# baseline (speedup 1.0000x reference)
import functools

import jax
import jax.numpy as jnp
from jax import lax
from jax.experimental import pallas as pl
from jax.experimental.pallas import tpu as pltpu
from jax.experimental.pallas import tpu_sc as plsc

N = 10000
D = 128
E = 320000
K = 112
LANES = 16
GROUPS = D // LANES
PAIRS = D // 32

_PERM = [32 * g + (k // 2 if k % 2 == 0 else 16 + k // 2)
         for g in range(PAIRS) for k in range(32)]


HALF = D // 2


def _sc_aggregate_fn(nc, ns, cpw0, cpw1):
    rows_per_tile = -(-N // (ns * 8)) * 8
    n_pad = ns * rows_per_tile
    e0 = ns * cpw0 * K

    mesh = plsc.VectorSubcoreMesh(core_axis_name="c", subcore_axis_name="s",
                                  num_cores=nc, num_subcores=ns)

    @functools.partial(
        pl.kernel,
        out_type=jax.ShapeDtypeStruct((2 * n_pad, D), jnp.float32),
        mesh=mesh,
        scratch_types=[
            pltpu.VMEM((K,), jnp.int32),
            pltpu.VMEM((K,), jnp.int32),
            pltpu.VMEM((K,), jnp.int32),
            pltpu.VMEM((K,), jnp.int32),
            pltpu.VMEM((K,), jnp.int32),
            pltpu.VMEM((K,), jnp.int32),
            pltpu.VMEM((K,), jnp.float32),
            pltpu.VMEM((K,), jnp.float32),
            pltpu.VMEM((K,), jnp.float32),
            pltpu.VMEM((K, D), jnp.float32),
            pltpu.VMEM((K, D), jnp.float32),
            pltpu.VMEM((K, D), jnp.float32),
            pltpu.VMEM_SHARED((n_pad, D), jnp.float32),
            pltpu.SemaphoreType.DMA,
            pltpu.SemaphoreType.DMA,
            pltpu.SemaphoreType.DMA,
            pltpu.SemaphoreType.DMA,
            pltpu.SemaphoreType.DMA,
            pltpu.SemaphoreType.DMA,
        ],
    )
    def agg(x_hbm, src_hbm, dst_hbm, w_hbm, out_hbm,
            src_0, src_1, src_2, dst_0, dst_1, dst_2, w_0, w_1, w_2,
            rows_0, rows_1, rows_2, acc, g0, g1, g2, s0, s1, s2):
        srcs = (src_0, src_1, src_2)
        dsts = (dst_0, dst_1, dst_2)
        ws = (w_0, w_1, w_2)
        rows = (rows_0, rows_1, rows_2)
        gsem = (g0, g1, g2)
        ssem = (s0, s1, s2)
        cid = lax.axis_index("c")
        sid = lax.axis_index("s")
        my_cpw = lax.select(cid == 0, cpw0, cpw1)
        base = lax.select(cid == 0, sid * (cpw0 * K), e0 + sid * (cpw1 * K))
        row0 = sid * rows_per_tile

        @pl.loop(0, K)
        def _zero_buf(i):
            for j in range(GROUPS):
                rows_0[i, pl.ds(j * LANES, LANES)] = jnp.zeros((LANES,), jnp.float32)

        done = 0
        while done < rows_per_tile:
            n = min(K, rows_per_tile - done)
            pltpu.sync_copy(rows_0.at[pl.ds(0, n)], acc.at[pl.ds(row0 + done, n)])
            done += n
        plsc.subcore_barrier()

        def load_idx(c, b):
            off = base + c * K
            pltpu.sync_copy(src_hbm.at[pl.ds(off, K)], srcs[b])
            pltpu.sync_copy(dst_hbm.at[pl.ds(off, K)], dsts[b])
            pltpu.sync_copy(w_hbm.at[pl.ds(off, K)], ws[b])

        def scale(t):
            w_ref = ws[t]
            rows_v = rows[t]

            @pl.loop(0, K // LANES, unroll=2)
            def _scale(ii):
                wv = w_ref[pl.ds(ii * LANES, LANES)]
                for l in range(LANES):
                    w = wv[l]
                    i = ii * LANES + l
                    for j in range(GROUPS):
                        sl = pl.ds(j * LANES, LANES)
                        rows_v[i, sl] = rows_v[i, sl] * w

        def gather(t):
            pltpu.async_copy(x_hbm.at[srcs[t]], rows[t], gsem[t])

        def scatter(t):
            pltpu.async_copy(rows[t], acc.at[dsts[t]], ssem[t], add=True)

        def wait_gather(t):
            pltpu.make_async_copy(x_hbm.at[srcs[t]], rows[t], gsem[t]).wait()

        def wait_scatter(t):
            pltpu.make_async_copy(rows[t], acc.at[dsts[t]], ssem[t]).wait()

        for t in range(3):
            load_idx(t, t)
            gather(t)

        @pl.loop(0, my_cpw, step=3)
        def _chunks(cc):
            for t in range(3):
                wait_gather(t)
                scale(t)
                scatter(t)

            for t in range(3):
                @pl.when(cc + t + 3 < my_cpw)
                def _(t=t):
                    wait_scatter(t)
                    load_idx(cc + t + 3, t)
                    gather(t)

        for t in range(3):
            wait_scatter(t)
        plsc.subcore_barrier()

        pltpu.sync_copy(acc.at[pl.ds(row0, rows_per_tile)],
                        out_hbm.at[pl.ds(cid * n_pad + row0, rows_per_tile)])

    return agg, n_pad


def _dense_kernel(x_ref, p0_ref, p1_ref, w_ref, b_ref, g_ref, be_ref,
                  h_ref, y_ref):
    hi = x_ref[...] + p0_ref[...] + p1_ref[...]
    z = jnp.dot(hi, w_ref[...], preferred_element_type=jnp.float32) + b_ref[...]
    z = jnp.where(z >= 0, z, 0.01 * z)
    m = jnp.mean(z, axis=-1, keepdims=True)
    v = jnp.mean((z - m) ** 2, axis=-1, keepdims=True)
    h = (z - m) * lax.rsqrt(v + 1e-5) * g_ref[...] + be_ref[...]
    h_ref[...] = h
    nrm = jnp.sqrt(jnp.sum(h * h, axis=-1, keepdims=True))
    y_ref[...] = h / jnp.maximum(nrm, 1e-12)


def _dense_layer(x, p0, p1, W, b, g, be):
    blk = 2000
    grid = (N // blk,)
    row_spec = pl.BlockSpec((blk, D), lambda i: (i, 0))
    rep_spec = pl.BlockSpec((1, D), lambda i: (0, 0))
    return pl.pallas_call(
        _dense_kernel,
        grid=grid,
        in_specs=[row_spec, row_spec, row_spec,
                  pl.BlockSpec((D, D), lambda i: (0, 0)),
                  rep_spec, rep_spec, rep_spec],
        out_specs=[row_spec, row_spec],
        out_shape=[jax.ShapeDtypeStruct((N, D), jnp.float32),
                   jax.ShapeDtypeStruct((N, D), jnp.float32)],
    )(x, p0, p1, W, b.reshape(1, D), g.reshape(1, D), be.reshape(1, D))


def kernel(ego_embeddings, edge_index, edge_weight, W1, b1, g1, be1,
           W2, b2, g2, be2):
    info = plsc.get_sparse_core_info()
    nc, ns = info.num_cores, info.num_subcores
    tot_cpt = -(-E // (ns * K * 3)) * 3
    cpw0 = int(round(tot_cpt * 0.68 / 3)) * 3
    cpw1 = tot_cpt - cpw0
    e_pad = ns * tot_cpt * K

    src = edge_index[0]
    dst = edge_index[1]
    pad = e_pad - E
    if pad:
        src = jnp.concatenate([src, jnp.zeros((pad,), jnp.int32)])
        dst = jnp.concatenate([dst, jnp.zeros((pad,), jnp.int32)])
        edge_weight = jnp.concatenate([edge_weight, jnp.zeros((pad,), jnp.float32)])

    agg, n_pad = _sc_aggregate_fn(nc, ns, cpw0, cpw1)

    def layer(x, W, b, g, be):
        part = agg(x, src, dst, edge_weight)
        return _dense_layer(x, part[:N], part[n_pad:n_pad + N], W, b, g, be)

    h1, y1 = layer(ego_embeddings, W1, b1, g1, be1)
    _, y2 = layer(h1, W2, b2, g2, be2)
    return jnp.concatenate([ego_embeddings, y1, y2], axis=1)
